# baseline (speedup 1.0000x reference)
import functools

import jax
import jax.numpy as jnp
from jax import lax
from jax.experimental import pallas as pl
from jax.experimental.pallas import tpu as pltpu

CROP = 14
C = 256
LEVEL_HW = [(64, 64), (32, 32), (16, 16), (8, 8), (4, 4)]
HMAX, WMAX = 64, 64
EPS = 1e-7
YPAD = HMAX + 1
KX = sum(w for _, w in LEVEL_HW)
NCOL = YPAD * C


ROWPAD = 16
XOFFS = [0, 64, 96, 112, 120]


def _tc_body(nb_blk, y0_sm, wy0_sm, wy1_sm, xp_ref,
             p0_ref, p1_ref, p2_ref, p3_ref, p4_ref,
             out_ref, ta_ref, tab_ref):
    i = pl.program_id(0)
    mp = nb_blk * ROWPAD

    @pl.when(i == 0)
    def _build():
        for lvl, p_ref in enumerate((p0_ref, p1_ref, p2_ref, p3_ref, p4_ref)):
            hl, wl = LEVEL_HW[lvl]
            xo = XOFFS[lvl]
            for y in range(hl):
                tab_ref[pl.ds(xo, wl), pl.ds(y * C, C)] = (
                    p_ref[y].astype(jnp.bfloat16))
            tab_ref[pl.ds(xo, wl), pl.ds(hl * C, (YPAD - hl) * C)] = (
                jnp.zeros((wl, (YPAD - hl) * C), jnp.bfloat16))
    xp = xp_ref[...]
    sx0 = xp[:, 0:1]
    sx1 = xp[:, 1:2]
    wx0 = xp[:, 2:3]
    wx1 = xp[:, 3:4]
    iota = lax.broadcasted_iota(jnp.int32, (mp, KX), 1).astype(jnp.float32)
    w_oh = (jnp.where(iota == sx0, wx0, 0.0)
            + jnp.where(iota == sx1, wx1, 0.0))
    ta_ref[...] = jnp.dot(w_oh.astype(jnp.bfloat16), tab_ref[...],
                          preferred_element_type=jnp.float32).astype(
                              jnp.bfloat16)
    for b in range(nb_blk):
        for j in range(CROP):
            g = i * (nb_blk * CROP) + b * CROP + j
            y0 = y0_sm[g]
            ystart = pl.multiple_of(y0 * C, C)
            s = ta_ref[pl.ds(b * ROWPAD, CROP),
                       pl.ds(ystart, 2 * C)].astype(jnp.float32)
            r = s[:, :C] * wy0_sm[g] + s[:, C:] * wy1_sm[g]
            for k in range(CROP):
                out_ref[0, j, k, pl.ds(b, 1)] = r[k:k + 1]


def _run(nb, bb, image_shape, boxes, p_list):
    img = image_shape.astype(jnp.float32)
    b = boxes[0]
    x1, y1, x2, y2 = b[:, 0], b[:, 1], b[:, 2], b[:, 3]
    w = x2 - x1
    h = y2 - y1
    size = jnp.sqrt(w * h)
    levels = jnp.floor(1.0 + jnp.log2(size / 224.0 + EPS))
    levels = jnp.clip(levels, 0.0, 4.0).astype(jnp.int32)
    Hs = jnp.array([hw[0] for hw in LEVEL_HW], dtype=jnp.float32)
    Ws = jnp.array([hw[1] for hw in LEVEL_HW], dtype=jnp.float32)
    fh = Hs[levels]
    fw = Ws[levels]
    y1n = y1 / img[1] * fh / (fh - 1.0)
    x1n = x1 / img[2] * fw / (fw - 1.0)
    y2n = (y2 / img[1] * fh - 1.0) / (fh - 1.0)
    x2n = (x2 / img[2] * fw - 1.0) / (fw - 1.0)
    i14 = jnp.arange(CROP, dtype=jnp.float32)
    ys = (y1n[:, None] * (fh[:, None] - 1.0)
          + i14[None, :] * (y2n - y1n)[:, None] * (fh[:, None] - 1.0) / (CROP - 1.0))
    xs = (x1n[:, None] * (fw[:, None] - 1.0)
          + i14[None, :] * (x2n - x1n)[:, None] * (fw[:, None] - 1.0) / (CROP - 1.0))
    valid_y = ((ys >= 0.0) & (ys <= fh[:, None] - 1.0)).astype(jnp.float32)
    valid_x = ((xs >= 0.0) & (xs <= fw[:, None] - 1.0)).astype(jnp.float32)
    y0f = jnp.floor(ys)
    x0f = jnp.floor(xs)
    ly = ys - y0f
    lx = xs - x0f
    y0i = jnp.clip(y0f, 0, HMAX - 1).astype(jnp.int32)
    x0i = jnp.clip(x0f, 0, WMAX - 1).astype(jnp.int32)
    x1i = jnp.clip(x0f + 1.0, 0, WMAX - 1).astype(jnp.int32)
    xoffv = jnp.array(XOFFS, dtype=jnp.int32)[levels]
    sx0 = xoffv[:, None] + x0i
    sx1 = xoffv[:, None] + x1i
    wy0 = (1.0 - ly) * valid_y
    wy1 = ly * valid_y
    wx0 = (1.0 - lx) * valid_x
    wx1 = lx * valid_x

    n_blk = -(-nb // bb)
    nbp = n_blk * bb
    bpad = nbp - nb
    mp = bb * ROWPAD
    xparams = jnp.stack(
        [sx0.astype(jnp.float32), sx1.astype(jnp.float32), wx0, wx1], axis=-1)
    xparams = jnp.pad(xparams, ((0, bpad), (0, ROWPAD - CROP), (0, 0)))
    xparams = xparams.reshape(nbp * ROWPAD, 4)
    y0p = jnp.pad(y0i, ((0, bpad), (0, 0))).reshape(-1)
    wy0p = jnp.pad(wy0, ((0, bpad), (0, 0))).reshape(-1)
    wy1p = jnp.pad(wy1, ((0, bpad), (0, 0))).reshape(-1)

    xin = lambda i, *_: (i, 0)
    grid_spec = pltpu.PrefetchScalarGridSpec(
        num_scalar_prefetch=3,
        grid=(n_blk,),
        in_specs=[
            pl.BlockSpec((mp, 4), xin),
        ] + [
            pl.BlockSpec(
                (LEVEL_HW[l][0], LEVEL_HW[l][1], C),
                lambda i, *_: (0, 0, 0))
            for l in range(5)
        ],
        out_specs=pl.BlockSpec((1, CROP, CROP, bb, C),
                               lambda i, *_: (0, 0, 0, i, 0)),
        scratch_shapes=[pltpu.VMEM((mp, NCOL), jnp.bfloat16),
                        pltpu.VMEM((KX, NCOL), jnp.bfloat16)],
    )
    out_t = pl.pallas_call(
        functools.partial(_tc_body, bb),
        grid_spec=grid_spec,
        out_shape=jax.ShapeDtypeStruct((1, CROP, CROP, nb, C), jnp.float32),
    )(y0p, wy0p, wy1p, xparams,
      p_list[0][0], p_list[1][0], p_list[2][0], p_list[3][0], p_list[4][0])
    return out_t.transpose(0, 3, 1, 2, 4)


def kernel(image_shape, boxes, scores, p0, p1, p2, p3, p4):
    del scores
    return _run(boxes.shape[1], 16, image_shape, boxes, (p0, p1, p2, p3, p4))


# speedup vs baseline: 1.0654x; 1.0654x over previous
import functools

import jax
import jax.numpy as jnp
from jax import lax
from jax.experimental import pallas as pl
from jax.experimental.pallas import tpu as pltpu

CROP = 14
C = 256
LEVEL_HW = [(64, 64), (32, 32), (16, 16), (8, 8), (4, 4)]
HMAX, WMAX = 64, 64
EPS = 1e-7
YPAD = HMAX + 1
KX = sum(w for _, w in LEVEL_HW)
NCOL = YPAD * C


ROWPAD = 16
XOFFS = [0, 64, 96, 112, 120]


def _tc_body(nb_blk, y0_sm, wy0_sm, wy1_sm, xp_ref,
             p0_ref, p1_ref, p2_ref, p3_ref, p4_ref,
             out_ref, ta_ref, tab_ref):
    i = pl.program_id(0)
    mp = nb_blk * ROWPAD

    @pl.when(i == 0)
    def _build():
        for lvl, p_ref in enumerate((p0_ref, p1_ref, p2_ref, p3_ref, p4_ref)):
            hl, wl = LEVEL_HW[lvl]
            xo = XOFFS[lvl]
            for y in range(hl):
                tab_ref[pl.ds(xo, wl), pl.ds(y * C, C)] = (
                    p_ref[y].astype(jnp.bfloat16))
            tab_ref[pl.ds(xo, wl), pl.ds(hl * C, (YPAD - hl) * C)] = (
                jnp.zeros((wl, (YPAD - hl) * C), jnp.bfloat16))
    xp = xp_ref[...]
    sx0 = xp[:, 0:1]
    sx1 = xp[:, 1:2]
    wx0 = xp[:, 2:3]
    wx1 = xp[:, 3:4]
    iota = lax.broadcasted_iota(jnp.int32, (mp, KX), 1).astype(jnp.float32)
    w_oh = (jnp.where(iota == sx0, wx0, 0.0)
            + jnp.where(iota == sx1, wx1, 0.0))
    ta_ref[...] = jnp.dot(w_oh.astype(jnp.bfloat16), tab_ref[...],
                          preferred_element_type=jnp.float32).astype(
                              jnp.bfloat16)
    for b in range(nb_blk):
        for j in range(CROP):
            g = i * (nb_blk * CROP) + b * CROP + j
            y0 = y0_sm[g]
            ystart = pl.multiple_of(y0 * C, C)
            s = ta_ref[pl.ds(b * ROWPAD, CROP),
                       pl.ds(ystart, 2 * C)].astype(jnp.float32)
            r = s[:, :C] * wy0_sm[g] + s[:, C:] * wy1_sm[g]
            for k in range(CROP):
                out_ref[0, j, k, pl.ds(b, 1)] = r[k:k + 1]


def _run(nb, bb, image_shape, boxes, p_list):
    img = image_shape.astype(jnp.float32)
    b = boxes[0]
    x1, y1, x2, y2 = b[:, 0], b[:, 1], b[:, 2], b[:, 3]
    w = x2 - x1
    h = y2 - y1
    size = jnp.sqrt(w * h)
    levels = jnp.floor(1.0 + jnp.log2(size / 224.0 + EPS))
    levels = jnp.clip(levels, 0.0, 4.0).astype(jnp.int32)
    Hs = jnp.array([hw[0] for hw in LEVEL_HW], dtype=jnp.float32)
    Ws = jnp.array([hw[1] for hw in LEVEL_HW], dtype=jnp.float32)
    fh = Hs[levels]
    fw = Ws[levels]
    y1n = y1 / img[1] * fh / (fh - 1.0)
    x1n = x1 / img[2] * fw / (fw - 1.0)
    y2n = (y2 / img[1] * fh - 1.0) / (fh - 1.0)
    x2n = (x2 / img[2] * fw - 1.0) / (fw - 1.0)
    i14 = jnp.arange(CROP, dtype=jnp.float32)
    ys = (y1n[:, None] * (fh[:, None] - 1.0)
          + i14[None, :] * (y2n - y1n)[:, None] * (fh[:, None] - 1.0) / (CROP - 1.0))
    xs = (x1n[:, None] * (fw[:, None] - 1.0)
          + i14[None, :] * (x2n - x1n)[:, None] * (fw[:, None] - 1.0) / (CROP - 1.0))
    valid_y = ((ys >= 0.0) & (ys <= fh[:, None] - 1.0)).astype(jnp.float32)
    valid_x = ((xs >= 0.0) & (xs <= fw[:, None] - 1.0)).astype(jnp.float32)
    y0f = jnp.floor(ys)
    x0f = jnp.floor(xs)
    ly = ys - y0f
    lx = xs - x0f
    y0i = jnp.clip(y0f, 0, HMAX - 1).astype(jnp.int32)
    x0i = jnp.clip(x0f, 0, WMAX - 1).astype(jnp.int32)
    x1i = jnp.clip(x0f + 1.0, 0, WMAX - 1).astype(jnp.int32)
    xoffv = jnp.array(XOFFS, dtype=jnp.int32)[levels]
    sx0 = xoffv[:, None] + x0i
    sx1 = xoffv[:, None] + x1i
    wy0 = (1.0 - ly) * valid_y
    wy1 = ly * valid_y
    wx0 = (1.0 - lx) * valid_x
    wx1 = lx * valid_x

    n_blk = -(-nb // bb)
    nbp = n_blk * bb
    bpad = nbp - nb
    mp = bb * ROWPAD
    xparams = jnp.stack(
        [sx0.astype(jnp.float32), sx1.astype(jnp.float32), wx0, wx1], axis=-1)
    xparams = jnp.pad(xparams, ((0, bpad), (0, ROWPAD - CROP), (0, 0)))
    xparams = xparams.reshape(nbp * ROWPAD, 4)
    y0p = jnp.pad(y0i, ((0, bpad), (0, 0))).reshape(-1)
    wy0p = jnp.pad(wy0, ((0, bpad), (0, 0))).reshape(-1)
    wy1p = jnp.pad(wy1, ((0, bpad), (0, 0))).reshape(-1)

    xin = lambda i, *_: (i, 0)
    grid_spec = pltpu.PrefetchScalarGridSpec(
        num_scalar_prefetch=3,
        grid=(n_blk,),
        in_specs=[
            pl.BlockSpec((mp, 4), xin),
        ] + [
            pl.BlockSpec(
                (LEVEL_HW[l][0], LEVEL_HW[l][1], C),
                lambda i, *_: (0, 0, 0))
            for l in range(5)
        ],
        out_specs=pl.BlockSpec((1, CROP, CROP, bb, C),
                               lambda i, *_: (0, 0, 0, i, 0)),
        scratch_shapes=[pltpu.VMEM((mp, NCOL), jnp.bfloat16),
                        pltpu.VMEM((KX, NCOL), jnp.bfloat16)],
    )
    out_t = pl.pallas_call(
        functools.partial(_tc_body, bb),
        grid_spec=grid_spec,
        out_shape=jax.ShapeDtypeStruct((1, CROP, CROP, nb, C), jnp.float32),
    )(y0p, wy0p, wy1p, xparams,
      p_list[0][0], p_list[1][0], p_list[2][0], p_list[3][0], p_list[4][0])
    return out_t.transpose(0, 3, 1, 2, 4)


def kernel(image_shape, boxes, scores, p0, p1, p2, p3, p4):
    del scores
    return _run(boxes.shape[1], 8, image_shape, boxes, (p0, p1, p2, p3, p4))


# speedup vs baseline: 1.0949x; 1.0277x over previous
import functools

import jax
import jax.numpy as jnp
from jax import lax
from jax.experimental import pallas as pl
from jax.experimental.pallas import tpu as pltpu

CROP = 14
C = 256
LEVEL_HW = [(64, 64), (32, 32), (16, 16), (8, 8), (4, 4)]
HMAX, WMAX = 64, 64
EPS = 1e-7
YPAD = HMAX + 1
KX = sum(w for _, w in LEVEL_HW)
NCOL = YPAD * C


ROWPAD = 16
XOFFS = [0, 64, 96, 112, 120]


def _tc_body(nb_blk, y0_sm, wy0_sm, wy1_sm, xp_ref,
             p0_ref, p1_ref, p2_ref, p3_ref, p4_ref,
             out_ref, ta_ref, tab_ref):
    i = pl.program_id(0)
    mp = nb_blk * ROWPAD

    @pl.when(i == 0)
    def _build():
        for lvl, p_ref in enumerate((p0_ref, p1_ref, p2_ref, p3_ref, p4_ref)):
            hl, wl = LEVEL_HW[lvl]
            xo = XOFFS[lvl]
            for y in range(hl):
                tab_ref[pl.ds(xo, wl), pl.ds(y * C, C)] = (
                    p_ref[y].astype(jnp.bfloat16))
            tab_ref[pl.ds(xo, wl), pl.ds(hl * C, (YPAD - hl) * C)] = (
                jnp.zeros((wl, (YPAD - hl) * C), jnp.bfloat16))
    xp = xp_ref[...]
    sx0 = xp[:, 0:1]
    sx1 = xp[:, 1:2]
    wx0 = xp[:, 2:3]
    wx1 = xp[:, 3:4]
    iota = lax.broadcasted_iota(jnp.int32, (mp, KX), 1).astype(jnp.float32)
    w_oh = (jnp.where(iota == sx0, wx0, 0.0)
            + jnp.where(iota == sx1, wx1, 0.0))
    ta_ref[...] = jnp.dot(w_oh.astype(jnp.bfloat16), tab_ref[...],
                          preferred_element_type=jnp.float32)
    for b in range(nb_blk):
        for j in range(CROP):
            g = i * (nb_blk * CROP) + b * CROP + j
            y0 = y0_sm[g]
            ystart = pl.multiple_of(y0 * C, C)
            s = ta_ref[pl.ds(b * ROWPAD, CROP), pl.ds(ystart, 2 * C)]
            r = s[:, :C] * wy0_sm[g] + s[:, C:] * wy1_sm[g]
            for k in range(CROP):
                out_ref[0, j, k, pl.ds(b, 1)] = r[k:k + 1]


def _run(nb, bb, image_shape, boxes, p_list):
    img = image_shape.astype(jnp.float32)
    b = boxes[0]
    x1, y1, x2, y2 = b[:, 0], b[:, 1], b[:, 2], b[:, 3]
    w = x2 - x1
    h = y2 - y1
    size = jnp.sqrt(w * h)
    levels = jnp.floor(1.0 + jnp.log2(size / 224.0 + EPS))
    levels = jnp.clip(levels, 0.0, 4.0).astype(jnp.int32)
    Hs = jnp.array([hw[0] for hw in LEVEL_HW], dtype=jnp.float32)
    Ws = jnp.array([hw[1] for hw in LEVEL_HW], dtype=jnp.float32)
    fh = Hs[levels]
    fw = Ws[levels]
    y1n = y1 / img[1] * fh / (fh - 1.0)
    x1n = x1 / img[2] * fw / (fw - 1.0)
    y2n = (y2 / img[1] * fh - 1.0) / (fh - 1.0)
    x2n = (x2 / img[2] * fw - 1.0) / (fw - 1.0)
    i14 = jnp.arange(CROP, dtype=jnp.float32)
    ys = (y1n[:, None] * (fh[:, None] - 1.0)
          + i14[None, :] * (y2n - y1n)[:, None] * (fh[:, None] - 1.0) / (CROP - 1.0))
    xs = (x1n[:, None] * (fw[:, None] - 1.0)
          + i14[None, :] * (x2n - x1n)[:, None] * (fw[:, None] - 1.0) / (CROP - 1.0))
    valid_y = ((ys >= 0.0) & (ys <= fh[:, None] - 1.0)).astype(jnp.float32)
    valid_x = ((xs >= 0.0) & (xs <= fw[:, None] - 1.0)).astype(jnp.float32)
    y0f = jnp.floor(ys)
    x0f = jnp.floor(xs)
    ly = ys - y0f
    lx = xs - x0f
    y0i = jnp.clip(y0f, 0, HMAX - 1).astype(jnp.int32)
    x0i = jnp.clip(x0f, 0, WMAX - 1).astype(jnp.int32)
    x1i = jnp.clip(x0f + 1.0, 0, WMAX - 1).astype(jnp.int32)
    xoffv = jnp.array(XOFFS, dtype=jnp.int32)[levels]
    sx0 = xoffv[:, None] + x0i
    sx1 = xoffv[:, None] + x1i
    wy0 = (1.0 - ly) * valid_y
    wy1 = ly * valid_y
    wx0 = (1.0 - lx) * valid_x
    wx1 = lx * valid_x

    n_blk = -(-nb // bb)
    nbp = n_blk * bb
    bpad = nbp - nb
    mp = bb * ROWPAD
    xparams = jnp.stack(
        [sx0.astype(jnp.float32), sx1.astype(jnp.float32), wx0, wx1], axis=-1)
    xparams = jnp.pad(xparams, ((0, bpad), (0, ROWPAD - CROP), (0, 0)))
    xparams = xparams.reshape(nbp * ROWPAD, 4)
    y0p = jnp.pad(y0i, ((0, bpad), (0, 0))).reshape(-1)
    wy0p = jnp.pad(wy0, ((0, bpad), (0, 0))).reshape(-1)
    wy1p = jnp.pad(wy1, ((0, bpad), (0, 0))).reshape(-1)

    xin = lambda i, *_: (i, 0)
    grid_spec = pltpu.PrefetchScalarGridSpec(
        num_scalar_prefetch=3,
        grid=(n_blk,),
        in_specs=[
            pl.BlockSpec((mp, 4), xin),
        ] + [
            pl.BlockSpec(
                (LEVEL_HW[l][0], LEVEL_HW[l][1], C),
                lambda i, *_: (0, 0, 0))
            for l in range(5)
        ],
        out_specs=pl.BlockSpec((1, CROP, CROP, bb, C),
                               lambda i, *_: (0, 0, 0, i, 0)),
        scratch_shapes=[pltpu.VMEM((mp, NCOL), jnp.float32),
                        pltpu.VMEM((KX, NCOL), jnp.bfloat16)],
    )
    out_t = pl.pallas_call(
        functools.partial(_tc_body, bb),
        grid_spec=grid_spec,
        out_shape=jax.ShapeDtypeStruct((1, CROP, CROP, nb, C), jnp.float32),
    )(y0p, wy0p, wy1p, xparams,
      p_list[0][0], p_list[1][0], p_list[2][0], p_list[3][0], p_list[4][0])
    return out_t.transpose(0, 3, 1, 2, 4)


def kernel(image_shape, boxes, scores, p0, p1, p2, p3, p4):
    del scores
    return _run(boxes.shape[1], 8, image_shape, boxes, (p0, p1, p2, p3, p4))
